# Initial kernel scaffold; baseline (speedup 1.0000x reference)
import jax
import jax.numpy as jnp
from jax.experimental import pallas as pl


def kernel(nodes, ent_features):
    raise NotImplementedError("write your pallas kernel here")


# speedup vs baseline: 8.1935x; 8.1935x over previous
import functools

import jax
import jax.numpy as jnp
from jax import lax
from jax.experimental import pallas as pl
from jax.experimental.pallas import tpu as pltpu
from jax.experimental.pallas import tpu_sc as plsc

D = 128
IDX_LANES = 128


@functools.partial(jax.jit, static_argnums=(2, 3))
def _sc_gather(idx_rows, table, n_rows, n_idx_rows):
    info = plsc.get_sparse_core_info()
    nc, ns = info.num_cores, info.num_subcores
    nw = nc * ns
    rows_per_w = n_idx_rows // nw
    K = 4
    n_iter = rows_per_w // K

    mesh = plsc.VectorSubcoreMesh(core_axis_name="c", subcore_axis_name="s")

    @functools.partial(
        pl.kernel,
        mesh=mesh,
        out_type=jax.ShapeDtypeStruct((n_idx_rows * IDX_LANES, D), jnp.float32),
        scratch_types=[
            pltpu.VMEM((K, IDX_LANES), jnp.int32),
            pltpu.VMEM((K * IDX_LANES, D), jnp.float32),
            pltpu.SemaphoreType.DMA,
        ],
    )
    def k(idx_hbm, table_hbm, out_hbm, idx_v, rows_v, sem):
        wid = lax.axis_index("s") * nc + lax.axis_index("c")
        row0 = wid * rows_per_w

        def body(i, carry):
            r = row0 + i * K
            pltpu.sync_copy(idx_hbm.at[pl.ds(r, K)], idx_v)
            cps = [
                pltpu.async_copy(
                    table_hbm.at[idx_v.at[j]],
                    rows_v.at[pl.ds(j * IDX_LANES, IDX_LANES)],
                    sem,
                )
                for j in range(K)
            ]
            for cp in cps:
                cp.wait()
            pltpu.sync_copy(
                rows_v, out_hbm.at[pl.ds(r * IDX_LANES, K * IDX_LANES)]
            )
            return carry

        lax.fori_loop(0, n_iter, body, 0)

    return k(idx_rows, table)


def kernel(nodes, ent_features):
    b, l = nodes.shape
    n = b * l
    idx_rows = nodes.reshape(n // IDX_LANES, IDX_LANES).astype(jnp.int32)
    out = _sc_gather(idx_rows, ent_features,
                     ent_features.shape[0], n // IDX_LANES)
    return out.reshape(b, l, D)


# speedup vs baseline: 9.1690x; 1.1191x over previous
import functools

import jax
import jax.numpy as jnp
from jax import lax
from jax.experimental import pallas as pl
from jax.experimental.pallas import tpu as pltpu
from jax.experimental.pallas import tpu_sc as plsc

D = 128
IDX_LANES = 128
K = 2
CHUNK = K * IDX_LANES


@functools.partial(jax.jit, static_argnums=(2, 3))
def _sc_gather(idx_rows, table, n_rows, n_idx_rows):
    info = plsc.get_sparse_core_info()
    nc, ns = info.num_cores, info.num_subcores
    nw = nc * ns
    rows_per_w = n_idx_rows // nw
    n_iter = rows_per_w // K

    mesh = plsc.VectorSubcoreMesh(core_axis_name="c", subcore_axis_name="s")

    @functools.partial(
        pl.kernel,
        mesh=mesh,
        out_type=jax.ShapeDtypeStruct((n_idx_rows * IDX_LANES, D), jnp.float32),
        scratch_types=[
            pltpu.VMEM((rows_per_w, IDX_LANES), jnp.int32),
            pltpu.VMEM((CHUNK, D), jnp.float32),
            pltpu.VMEM((CHUNK, D), jnp.float32),
            pltpu.SemaphoreType.DMA,
            pltpu.SemaphoreType.DMA,
            pltpu.SemaphoreType.DMA,
            pltpu.SemaphoreType.DMA,
        ],
    )
    def k(idx_hbm, table_hbm, out_hbm, idx_v, rows0, rows1, g0, g1, o0, o1):
        wid = lax.axis_index("s") * nc + lax.axis_index("c")
        row0 = wid * rows_per_w
        out0 = row0 * IDX_LANES
        rows = (rows0, rows1)
        gsem = (g0, g1)
        osem = (o0, o1)

        pltpu.sync_copy(idx_hbm.at[pl.ds(row0, rows_per_w)], idx_v)

        def fire(slot, it):
            for j in range(K):
                pltpu.async_copy(
                    table_hbm.at[idx_v.at[it * K + j]],
                    rows[slot].at[pl.ds(j * IDX_LANES, IDX_LANES)],
                    gsem[slot],
                )

        def drain_gather(slot):
            for j in range(K):
                pltpu.make_async_copy(
                    table_hbm.at[idx_v.at[j]],
                    rows[slot].at[pl.ds(j * IDX_LANES, IDX_LANES)],
                    gsem[slot],
                ).wait()

        def drain_out(slot):
            pltpu.make_async_copy(
                out_hbm.at[pl.ds(0, CHUNK)], rows[slot], osem[slot]
            ).wait()

        fire(0, 0)

        def body(g, carry):
            for b in range(2):
                it = 2 * g + b
                nb = 1 - b
                drain_gather(b)
                pltpu.async_copy(
                    rows[b], out_hbm.at[pl.ds(out0 + it * CHUNK, CHUNK)], osem[b]
                )

                @pl.when(it >= 1)
                def _():
                    drain_out(nb)

                @pl.when(it + 1 < n_iter)
                def _():
                    fire(nb, it + 1)

            return carry

        lax.fori_loop(0, n_iter // 2, body, 0)
        drain_out(1)

    return k(idx_rows, table)


def kernel(nodes, ent_features):
    b, l = nodes.shape
    n = b * l
    idx_rows = nodes.reshape(n // IDX_LANES, IDX_LANES).astype(jnp.int32)
    out = _sc_gather(idx_rows, ent_features,
                     ent_features.shape[0], n // IDX_LANES)
    return out.reshape(b, l, D)


# speedup vs baseline: 9.1974x; 1.0031x over previous
import functools

import jax
import jax.numpy as jnp
from jax import lax
from jax.experimental import pallas as pl
from jax.experimental.pallas import tpu as pltpu
from jax.experimental.pallas import tpu_sc as plsc

D = 128
IDX_LANES = 128
K = 2
CHUNK = K * IDX_LANES


@functools.partial(jax.jit, static_argnums=(2, 3))
def _sc_gather(idx_rows, table, n_rows, n_idx_rows):
    info = plsc.get_sparse_core_info()
    nc, ns = info.num_cores, info.num_subcores
    nw = nc * ns
    rows_per_w = n_idx_rows // nw
    n_iter = rows_per_w // K

    mesh = plsc.VectorSubcoreMesh(core_axis_name="c", subcore_axis_name="s")

    @functools.partial(
        pl.kernel,
        mesh=mesh,
        out_type=jax.ShapeDtypeStruct((n_idx_rows * IDX_LANES, D), jnp.float32),
        scratch_types=[
            pltpu.VMEM((rows_per_w, IDX_LANES), jnp.int32),
            pltpu.VMEM((CHUNK, D), jnp.float32),
            pltpu.VMEM((CHUNK, D), jnp.float32),
            pltpu.SemaphoreType.DMA,
            pltpu.SemaphoreType.DMA,
            pltpu.SemaphoreType.DMA,
            pltpu.SemaphoreType.DMA,
        ],
    )
    def k(idx_hbm, table_hbm, out_hbm, idx_v, rows0, rows1, g0, g1, o0, o1):
        wid = lax.axis_index("s") * nc + lax.axis_index("c")
        row0 = wid * rows_per_w
        out0 = row0 * IDX_LANES
        rows = (rows0, rows1)
        gsem = (g0, g1)
        osem = (o0, o1)

        pltpu.sync_copy(idx_hbm.at[pl.ds(row0, rows_per_w)], idx_v)

        def fire(slot, it):
            for j in range(K):
                pltpu.async_copy(
                    table_hbm.at[idx_v.at[it * K + j]],
                    rows[slot].at[pl.ds(j * IDX_LANES, IDX_LANES)],
                    gsem[slot],
                )

        def drain_gather(slot):
            for j in range(K):
                pltpu.make_async_copy(
                    table_hbm.at[idx_v.at[j]],
                    rows[slot].at[pl.ds(j * IDX_LANES, IDX_LANES)],
                    gsem[slot],
                ).wait()

        def drain_out(slot):
            pltpu.make_async_copy(
                out_hbm.at[pl.ds(0, CHUNK)], rows[slot], osem[slot]
            ).wait()

        fire(0, 0)

        def body(g, carry):
            for b in range(2):
                it = 2 * g + b
                nb = 1 - b

                @pl.when(it >= 1)
                def _():
                    drain_out(nb)

                @pl.when(it + 1 < n_iter)
                def _():
                    fire(nb, it + 1)

                drain_gather(b)
                pltpu.async_copy(
                    rows[b], out_hbm.at[pl.ds(out0 + it * CHUNK, CHUNK)], osem[b]
                )

            return carry

        lax.fori_loop(0, n_iter // 2, body, 0)
        drain_out(1)

    return k(idx_rows, table)


def kernel(nodes, ent_features):
    b, l = nodes.shape
    n = b * l
    idx_rows = nodes.reshape(n // IDX_LANES, IDX_LANES).astype(jnp.int32)
    out = _sc_gather(idx_rows, ent_features,
                     ent_features.shape[0], n // IDX_LANES)
    return out.reshape(b, l, D)
